# baseline (speedup 1.0000x reference)
import functools

import jax
import jax.numpy as jnp
from jax import lax
from jax.experimental import pallas as pl
from jax.experimental.pallas import tpu as pltpu
from jax.experimental.pallas import tpu_sc as plsc

NUM_TABLES = 4
NUM_EMB = 1000000
EMB_DIM = 64
BATCH = 4096
L = 20

LANES = 16
NC = 2
NS = 16
NW = NC * NS

WORKERS_PER_TABLE = NW // NUM_TABLES
B_PER_W = BATCH // WORKERS_PER_TABLE
IDX_PER_W = B_PER_W * L
BAGS_PER_CHUNK = 4
ROWS_PER_CHUNK = BAGS_PER_CHUNK * L
NUM_CHUNKS = B_PER_W // BAGS_PER_CHUNK


def _body(idx_hbm, tab_hbm, out_hbm, idx_v, buf0, buf1, out_v, sem0, sem1):
    cid = lax.axis_index("c")
    sid = lax.axis_index("s")
    wid = cid * NS + sid
    t = wid // WORKERS_PER_TABLE
    bstart = (wid % WORKERS_PER_TABLE) * B_PER_W

    pltpu.sync_copy(idx_hbm.at[pl.ds(wid * IDX_PER_W, IDX_PER_W)], idx_v)

    off = jnp.full((LANES,), t * NUM_EMB, dtype=jnp.int32)

    def add_off(i, _):
        sl = pl.ds(i * LANES, LANES)
        idx_v[sl] = idx_v[sl] + off
        return 0

    lax.fori_loop(0, IDX_PER_W // LANES, add_off, 0)

    bufs = (buf0, buf1)
    sems = (sem0, sem1)

    def start(g, buf, sem):
        pltpu.make_async_copy(
            tab_hbm.at[idx_v.at[pl.ds(g * ROWS_PER_CHUNK, ROWS_PER_CHUNK)]],
            buf, sem).start()

    def wait(g, buf, sem):
        pltpu.make_async_copy(
            tab_hbm.at[idx_v.at[pl.ds(g * ROWS_PER_CHUNK, ROWS_PER_CHUNK)]],
            buf, sem).wait()

    start(0, buf0, sem0)
    start(1, buf1, sem1)

    def reduce_chunk(g, buf):
        for j in range(BAGS_PER_CHUNK):
            for c in range(EMB_DIM // LANES):
                cs = pl.ds(c * LANES, LANES)
                acc = buf[j * L, cs]
                for l in range(1, L):
                    acc = acc + buf[j * L + l, cs]
                out_v[g * BAGS_PER_CHUNK + j, cs] = acc

    def loop_body(i, _):
        for b in range(2):
            g = i * 2 + b
            wait(g, bufs[b], sems[b])
            reduce_chunk(g, bufs[b])
            nxt = g + 2

            @pl.when(nxt < NUM_CHUNKS)
            def _():
                start(nxt, bufs[b], sems[b])
        return 0

    lax.fori_loop(0, NUM_CHUNKS // 2, loop_body, 0)

    pltpu.sync_copy(out_v, out_hbm.at[pl.ds(bstart, B_PER_W), t])


@jax.jit
def kernel(indices, tables):
    idx_flat = indices.reshape(-1)
    tab_flat = tables.reshape(NUM_TABLES * NUM_EMB, EMB_DIM)
    mesh = plsc.VectorSubcoreMesh(
        core_axis_name="c", subcore_axis_name="s",
        num_cores=NC, num_subcores=NS)
    out = pl.kernel(
        _body,
        out_type=jax.ShapeDtypeStruct((BATCH, NUM_TABLES, EMB_DIM),
                                      jnp.float32),
        mesh=mesh,
        scratch_types=[
            pltpu.VMEM((IDX_PER_W,), jnp.int32),
            pltpu.VMEM((ROWS_PER_CHUNK, EMB_DIM), jnp.float32),
            pltpu.VMEM((ROWS_PER_CHUNK, EMB_DIM), jnp.float32),
            pltpu.VMEM((B_PER_W, EMB_DIM), jnp.float32),
            pltpu.SemaphoreType.DMA,
            pltpu.SemaphoreType.DMA,
        ],
        compiler_params=pltpu.CompilerParams(use_tc_tiling_on_sc=False),
    )(idx_flat, tab_flat)
    return out.reshape(BATCH, NUM_TABLES * EMB_DIM)


# speedup vs baseline: 1.0111x; 1.0111x over previous
import functools

import jax
import jax.numpy as jnp
from jax import lax
from jax.experimental import pallas as pl
from jax.experimental.pallas import tpu as pltpu
from jax.experimental.pallas import tpu_sc as plsc

NUM_TABLES = 4
NUM_EMB = 1000000
EMB_DIM = 64
BATCH = 4096
L = 20

LANES = 16
NC = 2
NS = 16
NW = NC * NS

WORKERS_PER_TABLE = NW // NUM_TABLES
B_PER_W = BATCH // WORKERS_PER_TABLE
IDX_PER_W = B_PER_W * L
BAGS_PER_CHUNK = 16
ROWS_PER_CHUNK = BAGS_PER_CHUNK * L
NUM_CHUNKS = B_PER_W // BAGS_PER_CHUNK


def _body(idx_hbm, tab_hbm, out_hbm, idx_v, buf0, buf1, out_v, sem0, sem1):
    cid = lax.axis_index("c")
    sid = lax.axis_index("s")
    wid = cid * NS + sid
    t = wid // WORKERS_PER_TABLE
    bstart = (wid % WORKERS_PER_TABLE) * B_PER_W

    pltpu.sync_copy(idx_hbm.at[pl.ds(wid * IDX_PER_W, IDX_PER_W)], idx_v)

    off = jnp.full((LANES,), t * NUM_EMB, dtype=jnp.int32)

    def add_off(i, _):
        sl = pl.ds(i * LANES, LANES)
        idx_v[sl] = idx_v[sl] + off
        return 0

    lax.fori_loop(0, IDX_PER_W // LANES, add_off, 0)

    bufs = (buf0, buf1)
    sems = (sem0, sem1)

    def start(g, buf, sem):
        pltpu.make_async_copy(
            tab_hbm.at[idx_v.at[pl.ds(g * ROWS_PER_CHUNK, ROWS_PER_CHUNK)]],
            buf, sem).start()

    def wait(g, buf, sem):
        pltpu.make_async_copy(
            tab_hbm.at[idx_v.at[pl.ds(g * ROWS_PER_CHUNK, ROWS_PER_CHUNK)]],
            buf, sem).wait()

    start(0, buf0, sem0)
    start(1, buf1, sem1)

    def reduce_chunk(g, buf):
        def bag_body(j, _):
            for c in range(EMB_DIM // LANES):
                cs = pl.ds(c * LANES, LANES)
                acc = buf[j * L, cs]
                for l in range(1, L):
                    acc = acc + buf[j * L + l, cs]
                out_v[g * BAGS_PER_CHUNK + j, cs] = acc
            return 0

        lax.fori_loop(0, BAGS_PER_CHUNK, bag_body, 0)

    def loop_body(i, _):
        for b in range(2):
            g = i * 2 + b
            wait(g, bufs[b], sems[b])
            reduce_chunk(g, bufs[b])
            nxt = g + 2

            @pl.when(nxt < NUM_CHUNKS)
            def _():
                start(nxt, bufs[b], sems[b])
        return 0

    lax.fori_loop(0, NUM_CHUNKS // 2, loop_body, 0)

    pltpu.sync_copy(out_v, out_hbm.at[pl.ds(bstart, B_PER_W), t])


@jax.jit
def kernel(indices, tables):
    idx_flat = indices.reshape(-1)
    tab_flat = tables.reshape(NUM_TABLES * NUM_EMB, EMB_DIM)
    mesh = plsc.VectorSubcoreMesh(
        core_axis_name="c", subcore_axis_name="s",
        num_cores=NC, num_subcores=NS)
    out = pl.kernel(
        _body,
        out_type=jax.ShapeDtypeStruct((BATCH, NUM_TABLES, EMB_DIM),
                                      jnp.float32),
        mesh=mesh,
        scratch_types=[
            pltpu.VMEM((IDX_PER_W,), jnp.int32),
            pltpu.VMEM((ROWS_PER_CHUNK, EMB_DIM), jnp.float32),
            pltpu.VMEM((ROWS_PER_CHUNK, EMB_DIM), jnp.float32),
            pltpu.VMEM((B_PER_W, EMB_DIM), jnp.float32),
            pltpu.SemaphoreType.DMA,
            pltpu.SemaphoreType.DMA,
        ],
        compiler_params=pltpu.CompilerParams(use_tc_tiling_on_sc=False),
    )(idx_flat, tab_flat)
    return out.reshape(BATCH, NUM_TABLES * EMB_DIM)


# speedup vs baseline: 1.0144x; 1.0032x over previous
import functools

import jax
import jax.numpy as jnp
from jax import lax
from jax.experimental import pallas as pl
from jax.experimental.pallas import tpu as pltpu
from jax.experimental.pallas import tpu_sc as plsc

NUM_TABLES = 4
NUM_EMB = 1000000
EMB_DIM = 64
BATCH = 4096
L = 20

LANES = 16
NC = 2
NS = 16
NW = NC * NS

WORKERS_PER_TABLE = NW // NUM_TABLES
B_PER_W = BATCH // WORKERS_PER_TABLE
IDX_PER_W = B_PER_W * L
BAGS_PER_CHUNK = 16
ROWS_PER_CHUNK = BAGS_PER_CHUNK * L
NUM_CHUNKS = B_PER_W // BAGS_PER_CHUNK


def _body(idx_hbm, tab3_hbm, out_hbm, idx_v, buf0, buf1, out_v, sem0, sem1):
    cid = lax.axis_index("c")
    sid = lax.axis_index("s")
    wid = cid * NS + sid
    t = wid // WORKERS_PER_TABLE
    bstart = (wid % WORKERS_PER_TABLE) * B_PER_W
    tab_hbm = tab3_hbm.at[t]

    pltpu.sync_copy(idx_hbm.at[pl.ds(wid * IDX_PER_W, IDX_PER_W)], idx_v)

    bufs = (buf0, buf1)
    sems = (sem0, sem1)

    def start(g, buf, sem):
        pltpu.make_async_copy(
            tab_hbm.at[idx_v.at[pl.ds(g * ROWS_PER_CHUNK, ROWS_PER_CHUNK)]],
            buf, sem).start()

    def wait(g, buf, sem):
        pltpu.make_async_copy(
            tab_hbm.at[idx_v.at[pl.ds(g * ROWS_PER_CHUNK, ROWS_PER_CHUNK)]],
            buf, sem).wait()

    start(0, buf0, sem0)
    start(1, buf1, sem1)

    def reduce_chunk(g, buf):
        def bag_body(j, _):
            for c in range(EMB_DIM // LANES):
                cs = pl.ds(c * LANES, LANES)
                acc = buf[j * L, cs]
                for l in range(1, L):
                    acc = acc + buf[j * L + l, cs]
                out_v[g * BAGS_PER_CHUNK + j, cs] = acc
            return 0

        lax.fori_loop(0, BAGS_PER_CHUNK, bag_body, 0)

    def loop_body(i, _):
        for b in range(2):
            g = i * 2 + b
            wait(g, bufs[b], sems[b])
            reduce_chunk(g, bufs[b])
            nxt = g + 2

            @pl.when(nxt < NUM_CHUNKS)
            def _():
                start(nxt, bufs[b], sems[b])
        return 0

    lax.fori_loop(0, NUM_CHUNKS // 2, loop_body, 0)

    pltpu.sync_copy(out_v, out_hbm.at[pl.ds(bstart, B_PER_W), t])


@jax.jit
def kernel(indices, tables):
    idx_flat = indices.reshape(-1)
    mesh = plsc.VectorSubcoreMesh(
        core_axis_name="c", subcore_axis_name="s",
        num_cores=NC, num_subcores=NS)
    out = pl.kernel(
        _body,
        out_type=jax.ShapeDtypeStruct((BATCH, NUM_TABLES, EMB_DIM),
                                      jnp.float32),
        mesh=mesh,
        scratch_types=[
            pltpu.VMEM((IDX_PER_W,), jnp.int32),
            pltpu.VMEM((ROWS_PER_CHUNK, EMB_DIM), jnp.float32),
            pltpu.VMEM((ROWS_PER_CHUNK, EMB_DIM), jnp.float32),
            pltpu.VMEM((B_PER_W, EMB_DIM), jnp.float32),
            pltpu.SemaphoreType.DMA,
            pltpu.SemaphoreType.DMA,
        ],
        compiler_params=pltpu.CompilerParams(use_tc_tiling_on_sc=False),
    )(idx_flat, tables)
    return out.reshape(BATCH, NUM_TABLES * EMB_DIM)
